# Initial kernel scaffold; baseline (speedup 1.0000x reference)
import jax
import jax.numpy as jnp
from jax.experimental import pallas as pl


def kernel(x, edge_index, Wl1, Wr1, att1, b1, g1, be1, Wl2, Wr2, att2, b2, g2, be2):
    raise NotImplementedError("write your pallas kernel here")


# speedup vs baseline: 15.0173x; 15.0173x over previous
import functools

import jax
import jax.numpy as jnp
from jax import lax
from jax.experimental import pallas as pl
from jax.experimental.pallas import tpu as pltpu
from jax.experimental.pallas import tpu_sc as plsc

F32 = jnp.float32
BF16 = jnp.bfloat16
I32 = jnp.int32

NC = 2
NS = 16
NW = NC * NS
L = 16

_mesh = functools.partial(
    plsc.VectorSubcoreMesh,
    core_axis_name="c",
    subcore_axis_name="s",
    num_cores=NC,
    num_subcores=NS,
)

_sc_params = pltpu.CompilerParams(
    use_tc_tiling_on_sc=False, needs_layout_passes=False)


def _zero_rows(zref, nrows, ncols):
  zv = jnp.zeros((L,), F32)

  def body(i, carry):
    for j in range(ncols // L):
      zref[i, pl.ds(j * L, L)] = zv
    return carry

  lax.fori_loop(0, nrows, body, 0)


def _iota16():
  return lax.iota(I32, L)


def _make_logits_kernel(n, ept, e2, heads, cdim):
  nblk = ept // L
  assert nblk % 2 == 0
  npair = nblk // 2
  nchunk = 17 if npair % 17 == 0 else 1
  cpair = npair // nchunk
  cedge = cpair * 2 * L
  nq = cdim // (2 * L)
  rows_per_tile = n // NS

  def body(xl_hbm, xr_hbm, src_hbm, dst_hbm, att_hbm, ex_hbm, den_hbm,
           src_v, dst_v, att_v, xi0, xi1, xj0, xj1, m_v, dg0, dg1, exst_v,
           den_sh, si0, si1, sj0, sj1, sa0, sa1):
    cid = lax.axis_index("c")
    sid = lax.axis_index("s")
    wid = sid * NC + cid
    tile_base = wid * ept

    _zero_rows(dg0, L, 16)
    _zero_rows(dg1, L, 16)
    for j in range(rows_per_tile // L):
      pltpu.sync_copy(dg0, den_sh.at[pl.ds(sid * rows_per_tile + j * L, L)])
    rem = rows_per_tile % L
    if rem:
      pltpu.sync_copy(
          dg0.at[pl.ds(0, rem)],
          den_sh.at[pl.ds(sid * rows_per_tile + rows_per_tile - rem, rem)])
    plsc.subcore_barrier()

    pltpu.sync_copy(src_hbm.at[pl.ds(tile_base, ept)], src_v)
    pltpu.sync_copy(dst_hbm.at[pl.ds(tile_base, ept)], dst_v)
    pltpu.sync_copy(att_hbm, att_v)

    iota = _iota16()

    def compute(base, xi_v, xj_v, dg_v):
      cbase = lax.rem(base, cedge)
      valid = jnp.where(tile_base + base + iota < e2,
                        jnp.float32(1.0), jnp.float32(0.0))
      for h in range(heads):
        att_h = [att_v[h, pl.ds(g * 2 * L, 2 * L)] for g in range(nq)]
        for e in range(L):
          acc = jnp.zeros((L,), F32)
          for g in range(nq):
            off = h * cdim + g * 2 * L
            z = xi_v[e, pl.ds(off, 2 * L)] + xj_v[e, pl.ds(off, 2 * L)]
            zl = jnp.maximum(z, BF16(0.2) * z)
            t0, t1 = plsc.unpack(zl * att_h[g],
                                 format=plsc.PackFormat.INTERLEAVED,
                                 preferred_element_type=F32)
            acc = acc + t0 + t1
          plsc.store_scatter(m_v, [iota * L + e], acc)
        logit = m_v[pl.ds(0, L)]
        for p in range(1, L):
          logit = logit + m_v[pl.ds(p * L, L)]
        exh = jnp.exp(logit) * valid
        exst_v[h, pl.ds(cbase, L)] = exh
        plsc.store_scatter(dg_v, [iota, jnp.full((L,), h, I32)], exh)

    def pair(p, carry):
      base_a = p * 2 * L
      base_b = base_a + L
      base_c = jnp.minimum(base_a + 2 * L, ept - L)
      sidx_a = src_v[pl.ds(base_a, L)]
      didx_a = dst_v[pl.ds(base_a, L)]
      sidx_b = src_v[pl.ds(base_b, L)]
      didx_b = dst_v[pl.ds(base_b, L)]
      pltpu.async_copy(xl_hbm.at[sidx_b], xj1, sj1)
      pltpu.async_copy(xr_hbm.at[didx_b], xi1, si1)
      pltpu.make_async_copy(xl_hbm.at[sidx_a], xj0, sj0).wait()
      pltpu.make_async_copy(xr_hbm.at[didx_a], xi0, si0).wait()

      @pl.when(p > 0)
      def _():
        pltpu.make_async_copy(dg0, den_sh.at[didx_a], sa0).wait()

      compute(base_a, xi0, xj0, dg0)
      pltpu.async_copy(dg0, den_sh.at[didx_a], sa0, add=True)

      sidx_c = src_v[pl.ds(base_c, L)]
      didx_c = dst_v[pl.ds(base_c, L)]
      pltpu.async_copy(xl_hbm.at[sidx_c], xj0, sj0)
      pltpu.async_copy(xr_hbm.at[didx_c], xi0, si0)

      @pl.when(p > 0)
      def _():
        pltpu.make_async_copy(dg1, den_sh.at[didx_b], sa1).wait()

      pltpu.make_async_copy(xl_hbm.at[sidx_b], xj1, sj1).wait()
      pltpu.make_async_copy(xr_hbm.at[didx_b], xi1, si1).wait()
      compute(base_b, xi1, xj1, dg1)
      pltpu.async_copy(dg1, den_sh.at[didx_b], sa1, add=True)
      return carry

    def chunk(c, carry):
      lax.fori_loop(c * cpair, (c + 1) * cpair, pair, 0)
      for h in range(heads):
        pltpu.sync_copy(
            exst_v.at[h],
            ex_hbm.at[h, pl.ds(tile_base + c * cedge, cedge)])
      return carry

    sidx0 = src_v[pl.ds(0, L)]
    didx0 = dst_v[pl.ds(0, L)]
    pltpu.async_copy(xl_hbm.at[sidx0], xj0, sj0)
    pltpu.async_copy(xr_hbm.at[didx0], xi0, si0)
    lax.fori_loop(0, nchunk, chunk, 0)
    pltpu.make_async_copy(xl_hbm.at[sidx0], xj0, sj0).wait()
    pltpu.make_async_copy(xr_hbm.at[didx0], xi0, si0).wait()
    pltpu.make_async_copy(dg0, den_sh.at[didx0], sa0).wait()
    pltpu.make_async_copy(dg1, den_sh.at[didx0], sa1).wait()

    plsc.subcore_barrier()
    pltpu.sync_copy(
        den_sh.at[pl.ds(sid * rows_per_tile, rows_per_tile)],
        den_hbm.at[cid, pl.ds(sid * rows_per_tile, rows_per_tile), :])

  ep = ept * NW
  return pl.kernel(
      body,
      out_type=(
          jax.ShapeDtypeStruct((heads, ep), F32),
          jax.ShapeDtypeStruct((NC, n, 16), F32),
      ),
      mesh=_mesh(),
      compiler_params=_sc_params,
      scratch_types=[
          pltpu.VMEM((ept,), I32),
          pltpu.VMEM((ept,), I32),
          pltpu.VMEM((heads, cdim), BF16),
          pltpu.VMEM((L, heads * cdim), BF16),
          pltpu.VMEM((L, heads * cdim), BF16),
          pltpu.VMEM((L, heads * cdim), BF16),
          pltpu.VMEM((L, heads * cdim), BF16),
          pltpu.VMEM((L * L,), F32),
          pltpu.VMEM((L, 16), F32),
          pltpu.VMEM((L, 16), F32),
          pltpu.VMEM((heads, cedge), F32),
          pltpu.VMEM_SHARED((n, 16), F32),
          pltpu.SemaphoreType.DMA,
          pltpu.SemaphoreType.DMA,
          pltpu.SemaphoreType.DMA,
          pltpu.SemaphoreType.DMA,
          pltpu.SemaphoreType.DMA,
          pltpu.SemaphoreType.DMA,
      ],
  )


def _make_aggregate_kernel(n, ept, heads, cdim, nchunk):
  nblk = ept // L
  assert nblk % (2 * nchunk) == 0
  cpair = nblk // (2 * nchunk)
  cedge = cpair * 2 * L
  nq = cdim // (2 * L)
  rows_per_tile = n // NS

  def body(xl_hbm, src_hbm, dst_hbm, ex_hbm, invd_hbm, out_hbm,
           srcc_v, dstc_v, exst_v, iv0, iv1, xj0, xj1, rw0, rw1,
           out_sh, sj0, sj1, sd0, sd1, sa0, sa1):
    cid = lax.axis_index("c")
    sid = lax.axis_index("s")
    wid = sid * NC + cid
    tile_base = wid * ept

    _zero_rows(rw0, L, cdim)
    for j in range(rows_per_tile // L):
      pltpu.sync_copy(rw0, out_sh.at[pl.ds(sid * rows_per_tile + j * L, L)])
    rem = rows_per_tile % L
    if rem:
      pltpu.sync_copy(
          rw0.at[pl.ds(0, rem)],
          out_sh.at[pl.ds(sid * rows_per_tile + rows_per_tile - rem, rem)])
    plsc.subcore_barrier()

    iota = _iota16()

    def compute(cbase, xj_v, iv_v, rw_v):
      alph = []
      for h in range(heads):
        invd_h = plsc.load_gather(iv_v, [iota, jnp.full((L,), h, I32)])
        alph.append(exst_v[h, pl.ds(cbase, L)] * invd_h)
      for e in range(L):
        a = [alph[h][e] for h in range(heads)]
        for g in range(nq):
          lo, hi = plsc.unpack(xj_v[e, pl.ds(g * 2 * L, 2 * L)],
                               format=plsc.PackFormat.INTERLEAVED,
                               preferred_element_type=F32)
          acc_lo = a[0] * lo
          acc_hi = a[0] * hi
          for h in range(1, heads):
            lo, hi = plsc.unpack(xj_v[e, pl.ds(h * cdim + g * 2 * L, 2 * L)],
                                 format=plsc.PackFormat.INTERLEAVED,
                                 preferred_element_type=F32)
            acc_lo = acc_lo + a[h] * lo
            acc_hi = acc_hi + a[h] * hi
          rw_v[e, pl.ds(g * 2 * L, L)] = acc_lo
          rw_v[e, pl.ds(g * 2 * L + L, L)] = acc_hi

    def pair(lp, carry):
      base_a = lp * 2 * L
      base_b = base_a + L
      base_c = jnp.minimum(base_a + 2 * L, cedge - L)
      sidx_b = srcc_v[pl.ds(base_b, L)]
      didx_b = dstc_v[pl.ds(base_b, L)]
      didx_a = dstc_v[pl.ds(base_a, L)]
      pltpu.async_copy(xl_hbm.at[sidx_b], xj1, sj1)
      pltpu.async_copy(invd_hbm.at[didx_b], iv1, sd1)
      pltpu.make_async_copy(xl_hbm.at[iota], xj0, sj0).wait()
      pltpu.make_async_copy(invd_hbm.at[iota], iv0, sd0).wait()

      @pl.when(lp > 0)
      def _():
        pltpu.make_async_copy(rw0, out_sh.at[iota], sa0).wait()

      compute(base_a, xj0, iv0, rw0)
      pltpu.async_copy(rw0, out_sh.at[didx_a], sa0, add=True)

      sidx_c = srcc_v[pl.ds(base_c, L)]
      didx_c = dstc_v[pl.ds(base_c, L)]
      pltpu.async_copy(xl_hbm.at[sidx_c], xj0, sj0)
      pltpu.async_copy(invd_hbm.at[didx_c], iv0, sd0)

      @pl.when(lp > 0)
      def _():
        pltpu.make_async_copy(rw1, out_sh.at[iota], sa1).wait()

      pltpu.make_async_copy(xl_hbm.at[iota], xj1, sj1).wait()
      pltpu.make_async_copy(invd_hbm.at[iota], iv1, sd1).wait()
      compute(base_b, xj1, iv1, rw1)
      pltpu.async_copy(rw1, out_sh.at[didx_b], sa1, add=True)
      return carry

    def drain_tail():
      pltpu.make_async_copy(xl_hbm.at[iota], xj0, sj0).wait()
      pltpu.make_async_copy(invd_hbm.at[iota], iv0, sd0).wait()
      pltpu.make_async_copy(rw0, out_sh.at[iota], sa0).wait()
      pltpu.make_async_copy(rw1, out_sh.at[iota], sa1).wait()

    def chunk(c, carry):
      @pl.when(c > 0)
      def _():
        drain_tail()
      pltpu.sync_copy(src_hbm.at[pl.ds(tile_base + c * cedge, cedge)],
                      srcc_v)
      pltpu.sync_copy(dst_hbm.at[pl.ds(tile_base + c * cedge, cedge)],
                      dstc_v)
      for h in range(heads):
        pltpu.sync_copy(
            ex_hbm.at[h, pl.ds(tile_base + c * cedge, cedge)],
            exst_v.at[h])
      sidx0 = srcc_v[pl.ds(0, L)]
      didx0 = dstc_v[pl.ds(0, L)]
      pltpu.async_copy(xl_hbm.at[sidx0], xj0, sj0)
      pltpu.async_copy(invd_hbm.at[didx0], iv0, sd0)
      lax.fori_loop(0, cpair, pair, 0)
      return carry

    lax.fori_loop(0, nchunk, chunk, 0)
    drain_tail()

    plsc.subcore_barrier()
    pltpu.sync_copy(
        out_sh.at[pl.ds(sid * rows_per_tile, rows_per_tile)],
        out_hbm.at[cid, pl.ds(sid * rows_per_tile, rows_per_tile), :])

  ep = ept * NW
  return pl.kernel(
      body,
      out_type=jax.ShapeDtypeStruct((NC, n, cdim), F32),
      mesh=_mesh(),
      compiler_params=_sc_params,
      scratch_types=[
          pltpu.VMEM((cedge,), I32),
          pltpu.VMEM((cedge,), I32),
          pltpu.VMEM((heads, cedge), F32),
          pltpu.VMEM((L, 16), F32),
          pltpu.VMEM((L, 16), F32),
          pltpu.VMEM((L, heads * cdim), BF16),
          pltpu.VMEM((L, heads * cdim), BF16),
          pltpu.VMEM((L, cdim), F32),
          pltpu.VMEM((L, cdim), F32),
          pltpu.VMEM_SHARED((n, cdim), F32),
          pltpu.SemaphoreType.DMA,
          pltpu.SemaphoreType.DMA,
          pltpu.SemaphoreType.DMA,
          pltpu.SemaphoreType.DMA,
          pltpu.SemaphoreType.DMA,
          pltpu.SemaphoreType.DMA,
      ],
  )


def _mm2_body(x_ref, wl_ref, wr_ref, xl_ref, xr_ref):
  xv = x_ref[...]
  xl_ref[...] = jnp.dot(
      xv, wl_ref[...], preferred_element_type=F32).astype(BF16)
  xr_ref[...] = jnp.dot(
      xv, wr_ref[...], preferred_element_type=F32).astype(BF16)


def _mm2(x, wl, wr, blk_rows):
  n, f = x.shape
  k = wl.shape[1]
  grid = n // blk_rows
  return pl.pallas_call(
      _mm2_body,
      grid=(grid,),
      in_specs=[
          pl.BlockSpec((blk_rows, f), lambda i: (i, 0)),
          pl.BlockSpec((f, k), lambda i: (0, 0)),
          pl.BlockSpec((f, k), lambda i: (0, 0)),
      ],
      out_specs=(
          pl.BlockSpec((blk_rows, k), lambda i: (i, 0)),
          pl.BlockSpec((blk_rows, k), lambda i: (i, 0)),
      ),
      out_shape=(
          jax.ShapeDtypeStruct((n, k), BF16),
          jax.ShapeDtypeStruct((n, k), BF16),
      ),
  )(x, wl, wr)


def _invden_body(denp_ref, out_ref, *, scale):
  d = denp_ref[0] + denp_ref[1]
  out_ref[...] = scale / (d + 1e-16)


def _invden(denp, scale):
  n = denp.shape[1]
  return pl.pallas_call(
      functools.partial(_invden_body, scale=scale),
      out_shape=jax.ShapeDtypeStruct((n, 16), F32),
  )(denp)


def _bn_relu(o, g, b):
  mu = jnp.mean(o, axis=0)
  var = jnp.mean(jnp.square(o - mu), axis=0)
  return jnp.maximum((o - mu) / jnp.sqrt(var + 1e-5) * g + b, 0.0)


def _bnmm_body(outp_ref, b1_ref, g1_ref, be1_ref, wl_ref, wr_ref,
               xl_ref, xr_ref):
  o = outp_ref[0] + outp_ref[1] + b1_ref[...][None, :]
  h = _bn_relu(o, g1_ref[...][None, :], be1_ref[...][None, :])
  xl_ref[...] = jnp.dot(
      h, wl_ref[...], preferred_element_type=F32).astype(BF16)
  xr_ref[...] = jnp.dot(
      h, wr_ref[...], preferred_element_type=F32).astype(BF16)


def _bnmm(outp, b1, g1, be1, wl, wr):
  n = outp.shape[1]
  k = wl.shape[1]
  return pl.pallas_call(
      _bnmm_body,
      out_shape=(
          jax.ShapeDtypeStruct((n, k), BF16),
          jax.ShapeDtypeStruct((n, k), BF16),
      ),
  )(outp, b1, g1, be1, wl, wr)


def _bnfinal_body(outp_ref, b2_ref, g2_ref, be2_ref, out_ref):
  o = outp_ref[0] + outp_ref[1] + b2_ref[...][None, :]
  out_ref[...] = _bn_relu(o, g2_ref[...][None, :], be2_ref[...][None, :])


def _bnfinal(outp, b2, g2, be2):
  n, k = outp.shape[1], outp.shape[2]
  return pl.pallas_call(
      _bnfinal_body,
      out_shape=jax.ShapeDtypeStruct((n, k), F32),
  )(outp, b2, g2, be2)


def kernel(x, edge_index, Wl1, Wr1, att1, b1, g1, be1, Wl2, Wr2, att2, b2,
           g2, be2):
  n, f_in = x.shape
  e = edge_index.shape[1]
  h1, hid = att1.shape
  out_dim = att2.shape[1]

  e2 = e + n
  ept = 2 * L * -(-e2 // (NW * 2 * L))
  ep = ept * NW
  loop_idx = jnp.arange(n, dtype=I32)
  pad = jnp.zeros((ep - e2,), I32)
  src = jnp.concatenate([edge_index[0].astype(I32), loop_idx, pad])
  dst = jnp.concatenate([edge_index[1].astype(I32), loop_idx, pad])

  def _ileave(width):
    cols = []
    for g in range(width // (2 * L)):
      for i in range(L):
        cols.extend([g * 2 * L + i, g * 2 * L + L + i])
    return jnp.array(cols, dtype=I32)

  p_hid = _ileave(hid)
  p1 = jnp.concatenate([p_hid + i * hid for i in range(h1)])
  p_out = _ileave(out_dim)
  Wl1 = Wl1[:, p1]
  Wr1 = Wr1[:, p1]
  att1 = att1[:, p_hid].astype(BF16)
  Wl2 = Wl2[:, p_out]
  Wr2 = Wr2[:, p_out]
  att2 = att2[:, p_out].astype(BF16)

  nchunk = 17 if (ept // (2 * L)) % 17 == 0 else 1
  logits1 = _make_logits_kernel(n, ept, e2, h1, hid)
  aggr1 = _make_aggregate_kernel(n, ept, h1, hid, nchunk)
  logits2 = _make_logits_kernel(n, ept, e2, 1, out_dim)
  aggr2 = _make_aggregate_kernel(n, ept, 1, out_dim, nchunk)

  xl1, xr1 = _mm2(x, Wl1, Wr1, 2000)
  ex1, denp1 = logits1(xl1, xr1, src, dst, att1)
  invd1 = _invden(denp1, 0.25)
  outp1 = aggr1(xl1, src, dst, ex1, invd1)

  xl2, xr2 = _bnmm(outp1, b1, g1, be1, Wl2, Wr2)

  ex2, denp2 = logits2(xl2, xr2, src, dst, att2)
  invd2 = _invden(denp2, 1.0)
  outp2 = aggr2(xl2, src, dst, ex2, invd2)

  return _bnfinal(outp2, b2, g2, be2)
